# Initial kernel scaffold; baseline (speedup 1.0000x reference)
import jax
import jax.numpy as jnp
from jax.experimental import pallas as pl


def kernel(item_id, brand, material, author, item_table, brand_table, material_table, author_table):
    raise NotImplementedError("write your pallas kernel here")


# speedup vs baseline: 6.1259x; 6.1259x over previous
import functools

import jax
import jax.numpy as jnp
from jax import lax
from jax.experimental import pallas as pl
from jax.experimental.pallas import tpu as pltpu
from jax.experimental.pallas import tpu_sc as plsc

B, L, D = 16384, 50, 64
N = B * L
NC, NS = 2, 16
NW = NC * NS
PER_W = N // NW
CH = 128
STEPS = PER_W // CH


def _body(it_t, br_t, ma_t, au_t, it_i, br_i, ma_i, au_i, out,
          i0, i1, i2, i3, r0, r1, r2, r3, sem):
    wid = lax.axis_index("s") * NC + lax.axis_index("c")
    w0 = wid * PER_W

    def step(g, carry):
        base = w0 + g * CH
        pltpu.sync_copy(it_i.at[pl.ds(base, CH)], i0)
        pltpu.sync_copy(br_i.at[pl.ds(base, CH)], i1)
        pltpu.sync_copy(ma_i.at[pl.ds(base, CH)], i2)
        pltpu.sync_copy(au_i.at[pl.ds(base, CH)], i3)
        c0 = pltpu.async_copy(it_t.at[i0], r0, sem)
        c1 = pltpu.async_copy(br_t.at[i1], r1, sem)
        c2 = pltpu.async_copy(ma_t.at[i2], r2, sem)
        c3 = pltpu.async_copy(au_t.at[i3], r3, sem)
        c0.wait()
        c1.wait()
        c2.wait()
        c3.wait()

        def row(rr, rcarry):
            for c in range(4):
                sl = pl.ds(c * 16, 16)
                r0[rr, sl] = (r0[rr, sl] + r1[rr, sl]) + (r2[rr, sl] + r3[rr, sl])
            return rcarry

        lax.fori_loop(0, CH, row, 0)
        pltpu.sync_copy(r0, out.at[pl.ds(base, CH)])
        return carry

    lax.fori_loop(0, STEPS, step, 0)


@jax.jit
def _run(item_id, brand, material, author, it_t, br_t, ma_t, au_t):
    mesh = plsc.VectorSubcoreMesh(core_axis_name="c", subcore_axis_name="s")
    k = pl.kernel(
        _body,
        mesh=mesh,
        out_type=jax.ShapeDtypeStruct((N, D), jnp.float32),
        compiler_params=pltpu.CompilerParams(use_tc_tiling_on_sc=False),
        scratch_types=[
            pltpu.VMEM((CH,), jnp.int32),
            pltpu.VMEM((CH,), jnp.int32),
            pltpu.VMEM((CH,), jnp.int32),
            pltpu.VMEM((CH,), jnp.int32),
            pltpu.VMEM((CH, D), jnp.float32),
            pltpu.VMEM((CH, D), jnp.float32),
            pltpu.VMEM((CH, D), jnp.float32),
            pltpu.VMEM((CH, D), jnp.float32),
            pltpu.SemaphoreType.DMA,
        ],
    )
    return k(it_t, br_t, ma_t, au_t,
             item_id.reshape(N), brand.reshape(N),
             material.reshape(N), author.reshape(N))


def kernel(item_id, brand, material, author, item_table, brand_table,
           material_table, author_table):
    out = _run(item_id, brand, material, author,
               item_table, brand_table, material_table, author_table)
    return out.reshape(B, L, D)


# speedup vs baseline: 8.8440x; 1.4437x over previous
import jax
import jax.numpy as jnp
from jax import lax
from jax.experimental import pallas as pl
from jax.experimental.pallas import tpu as pltpu
from jax.experimental.pallas import tpu_sc as plsc

B, L, D = 16384, 50, 64
N = B * L
NC, NS = 2, 16
NW = NC * NS
PER_W = N // NW
CH = 128
S = PER_W // CH
IB = 20
NBLK = S // IB


def _body(it_t, br_t, ma_t, au_t, ii, bi, mi, ai, out,
          ib_ref, r00, r01, r02, r03, r10, r11, r12, r13, ob0, ob1,
          semI, semG0, semG1, semW0, semW1):
    wid = lax.axis_index("s") * NC + lax.axis_index("c")
    row0 = wid * S
    w0 = wid * PER_W

    idx_hbms = (ii, bi, mi, ai)
    tabs = (it_t, br_t, ma_t, au_t)
    set0 = (r00, r01, r02, r03)
    set1 = (r10, r11, r12, r13)

    for t in range(4):
        pltpu.sync_copy(idx_hbms[t].at[pl.ds(row0, IB)],
                        ib_ref.at[t, pl.ds(0, IB)])
    for t in range(4):
        pltpu.async_copy(idx_hbms[t].at[pl.ds(row0 + IB, IB)],
                         ib_ref.at[t, pl.ds(IB, IB)], semI)
    for t in range(4):
        pltpu.async_copy(tabs[t].at[ib_ref.at[t, 0]], set0[t], semG0)

    def accumulate(rset, ob):
        def rowbody(rr, c_):
            for c in range(4):
                sl = pl.ds(c * 16, 16)
                ob[rr, sl] = ((rset[0][rr, sl] + rset[1][rr, sl])
                              + (rset[2][rr, sl] + rset[3][rr, sl]))
            return c_
        lax.fori_loop(0, CH, rowbody, 0)

    def dstep(g, carry):
        s0 = 2 * g
        blk = s0 // IB
        off = s0 - blk * IB
        slot = (blk % 2) * IB
        r_s1 = slot + off + 1
        blk2 = (s0 + 2) // IB
        r_s2 = ((blk2 % 2) * IB) + (s0 + 2 - blk2 * IB)

        for t in range(4):
            pltpu.async_copy(tabs[t].at[ib_ref.at[t, r_s1]], set1[t], semG1)

        @pl.when(jnp.logical_and(off == IB - 2, s0 + 2 < S))
        def _():
            for t in range(4):
                pltpu.make_async_copy(idx_hbms[t].at[pl.ds(row0, IB)],
                                      ib_ref.at[t, pl.ds(0, IB)], semI).wait()

        @pl.when(jnp.logical_and(off == 0,
                                 jnp.logical_and(s0 >= IB,
                                                 s0 < (NBLK - 1) * IB)))
        def _():
            nslot = ((blk + 1) % 2) * IB
            for t in range(4):
                pltpu.async_copy(
                    idx_hbms[t].at[pl.ds(row0 + (blk + 1) * IB, IB)],
                    ib_ref.at[t, pl.ds(nslot, IB)], semI)

        for t in range(4):
            pltpu.make_async_copy(it_t.at[pl.ds(0, CH)], set0[t], semG0).wait()

        @pl.when(g > 0)
        def _():
            pltpu.make_async_copy(ob0, out.at[pl.ds(w0, CH)], semW0).wait()

        accumulate(set0, ob0)
        pltpu.async_copy(ob0, out.at[pl.ds(w0 + s0 * CH, CH)], semW0)

        @pl.when(s0 + 2 < S)
        def _():
            for t in range(4):
                pltpu.async_copy(tabs[t].at[ib_ref.at[t, r_s2]], set0[t],
                                 semG0)

        for t in range(4):
            pltpu.make_async_copy(it_t.at[pl.ds(0, CH)], set1[t], semG1).wait()

        @pl.when(g > 0)
        def _():
            pltpu.make_async_copy(ob1, out.at[pl.ds(w0, CH)], semW1).wait()

        accumulate(set1, ob1)
        pltpu.async_copy(ob1, out.at[pl.ds(w0 + (s0 + 1) * CH, CH)], semW1)
        return carry

    lax.fori_loop(0, S // 2, dstep, 0)

    pltpu.make_async_copy(ob0, out.at[pl.ds(w0, CH)], semW0).wait()
    pltpu.make_async_copy(ob1, out.at[pl.ds(w0, CH)], semW1).wait()


@jax.jit
def _run(item_id, brand, material, author, it_t, br_t, ma_t, au_t):
    mesh = plsc.VectorSubcoreMesh(core_axis_name="c", subcore_axis_name="s")
    k = pl.kernel(
        _body,
        mesh=mesh,
        out_type=jax.ShapeDtypeStruct((N, D), jnp.float32),
        compiler_params=pltpu.CompilerParams(use_tc_tiling_on_sc=False),
        scratch_types=[
            pltpu.VMEM((4, 2 * IB, CH), jnp.int32),
            pltpu.VMEM((CH, D), jnp.float32),
            pltpu.VMEM((CH, D), jnp.float32),
            pltpu.VMEM((CH, D), jnp.float32),
            pltpu.VMEM((CH, D), jnp.float32),
            pltpu.VMEM((CH, D), jnp.float32),
            pltpu.VMEM((CH, D), jnp.float32),
            pltpu.VMEM((CH, D), jnp.float32),
            pltpu.VMEM((CH, D), jnp.float32),
            pltpu.VMEM((CH, D), jnp.float32),
            pltpu.VMEM((CH, D), jnp.float32),
            pltpu.SemaphoreType.DMA,
            pltpu.SemaphoreType.DMA,
            pltpu.SemaphoreType.DMA,
            pltpu.SemaphoreType.DMA,
            pltpu.SemaphoreType.DMA,
        ],
    )
    return k(it_t, br_t, ma_t, au_t,
             item_id.reshape(N // CH, CH), brand.reshape(N // CH, CH),
             material.reshape(N // CH, CH), author.reshape(N // CH, CH))


def kernel(item_id, brand, material, author, item_table, brand_table,
           material_table, author_table):
    out = _run(item_id, brand, material, author,
               item_table, brand_table, material_table, author_table)
    return out.reshape(B, L, D)
